# baseline (speedup 1.0000x reference)
import functools

import jax
import jax.numpy as jnp
from jax import lax
from jax.experimental import pallas as pl
from jax.experimental.pallas import tpu as pltpu
from jax.experimental.pallas import tpu_sc as plsc

NUM_EMB = 1_000_000
DIM = 32
BATCH = 16384
QUAD = 128
NUM_QROWS = NUM_EMB * DIM // QUAD

NUM_CORES = 2
NUM_SUBCORES = 16
NUM_WORKERS = NUM_CORES * NUM_SUBCORES
B_PER_W = BATCH // NUM_WORKERS
WAVE = 128
NWAVE = B_PER_W // WAVE
LANES = 16

_MESH = plsc.VectorSubcoreMesh(
    core_axis_name="c", subcore_axis_name="s",
    num_cores=NUM_CORES, num_subcores=NUM_SUBCORES)


@functools.partial(
    pl.kernel,
    out_type=jax.ShapeDtypeStruct((BATCH, DIM), jnp.float32),
    mesh=_MESH,
    scratch_types=[
        pltpu.VMEM((B_PER_W,), jnp.int32),
        pltpu.VMEM((B_PER_W,), jnp.int32),
        pltpu.VMEM((2, WAVE, QUAD), jnp.float32),
        pltpu.VMEM((2, WAVE, DIM), jnp.float32),
        pltpu.SemaphoreType.DMA,
        pltpu.SemaphoreType.DMA,
        pltpu.SemaphoreType.DMA,
        pltpu.SemaphoreType.DMA,
    ],
    compiler_params=pltpu.CompilerParams(
        use_tc_tiling_on_sc=True, needs_layout_passes=False),
)
def _gather(idx_hbm, tab_hbm, out_hbm, idx_s, q_s, blk_v, rows_v,
            g0, g1, o0, o1):
    wid = lax.axis_index("s") * NUM_CORES + lax.axis_index("c")
    base = pl.multiple_of(wid * B_PER_W, 8)
    pltpu.sync_copy(idx_hbm.at[pl.ds(base, B_PER_W)], idx_s)

    def qids(g, carry):
        v = idx_s[pl.ds(g * LANES, LANES)]
        q_s[pl.ds(g * LANES, LANES)] = v >> 2
        return carry

    lax.fori_loop(0, B_PER_W // LANES, qids, 0)

    gsem = (g0, g1)
    osem = (o0, o1)

    def fire(w, p):
        pltpu.async_copy(
            tab_hbm.at[q_s.at[pl.ds(w * WAVE, WAVE)]],
            blk_v.at[p], gsem[p])

    def wait_gather(p):
        pltpu.make_async_copy(
            tab_hbm.at[pl.ds(0, WAVE)], blk_v.at[p], gsem[p]).wait()

    def wait_out(p):
        pltpu.make_async_copy(
            out_hbm.at[pl.ds(0, WAVE), :], rows_v.at[p],
            osem[p]).wait()

    def extract(w, p):
        def body(g, carry):
            v = idx_s[pl.ds(w * WAVE + g * LANES, LANES)]
            off = (v & 3) << 5
            for k in range(LANES):
                i = g * LANES + k
                o = off[k]
                rows_v[p, i, pl.ds(0, LANES)] = (
                    blk_v[p, i, pl.ds(o, LANES)])
                rows_v[p, i, pl.ds(LANES, LANES)] = (
                    blk_v[p, i, pl.ds(o + LANES, LANES)])
            return carry

        lax.fori_loop(0, WAVE // LANES, body, 0)

    fire(0, 0)
    for w in range(NWAVE):
        p = w % 2
        if w + 1 < NWAVE:
            fire(w + 1, 1 - p)
        wait_gather(p)
        if w >= 2:
            wait_out(p)
        extract(w, p)
        pltpu.async_copy(rows_v.at[p],
                         out_hbm.at[pl.ds(base + w * WAVE, WAVE), :],
                         osem[p])
    wait_out(0)
    wait_out(1)


def kernel(indices, table):
    tabq = table.reshape(NUM_QROWS, QUAD)
    return _gather(indices.astype(jnp.int32), tabq)


# speedup vs baseline: 1.5226x; 1.5226x over previous
import functools

import jax
import jax.numpy as jnp
from jax import lax
from jax.experimental import pallas as pl
from jax.experimental.pallas import tpu as pltpu
from jax.experimental.pallas import tpu_sc as plsc

NUM_EMB = 1_000_000
DIM = 32
BATCH = 16384

NUM_CORES = 2
NUM_SUBCORES = 16
NUM_WORKERS = NUM_CORES * NUM_SUBCORES
B_PER_W = BATCH // NUM_WORKERS
WAVE = 32
NWAVE = B_PER_W // WAVE
BLK = 8
LANES = 16

_MESH = plsc.VectorSubcoreMesh(
    core_axis_name="c", subcore_axis_name="s",
    num_cores=NUM_CORES, num_subcores=NUM_SUBCORES)


@functools.partial(
    pl.kernel,
    out_type=jax.ShapeDtypeStruct((BATCH, DIM), jnp.float32),
    mesh=_MESH,
    scratch_types=[
        pltpu.VMEM((B_PER_W,), jnp.int32),
        pltpu.VMEM((2, WAVE * BLK, DIM), jnp.float32),
        pltpu.VMEM((2, WAVE, DIM), jnp.float32),
        pltpu.SemaphoreType.DMA,
        pltpu.SemaphoreType.DMA,
        pltpu.SemaphoreType.DMA,
        pltpu.SemaphoreType.DMA,
    ],
    compiler_params=pltpu.CompilerParams(
        use_tc_tiling_on_sc=True, needs_layout_passes=False),
)
def _gather(idx_hbm, tab_hbm, out_hbm, idx_s, blk_v, rows_v,
            g0, g1, o0, o1):
    wid = lax.axis_index("s") * NUM_CORES + lax.axis_index("c")
    base = pl.multiple_of(wid * B_PER_W, 8)
    pltpu.sync_copy(idx_hbm.at[pl.ds(base, B_PER_W)], idx_s)

    gsem = (g0, g1)
    osem = (o0, o1)

    def fire(w, p):
        def body(g, carry):
            v = idx_s[pl.ds(w * WAVE + g * LANES, LANES)]
            q = (v >> 3) * BLK
            for k in range(LANES):
                q8 = pl.multiple_of(q[k], BLK)
                pltpu.async_copy(
                    tab_hbm.at[pl.ds(q8, BLK), :],
                    blk_v.at[p, pl.ds((g * LANES + k) * BLK, BLK), :],
                    gsem[p])
            return carry

        lax.fori_loop(0, WAVE // LANES, body, 0)

    def wait_gather(p):
        pltpu.make_async_copy(
            tab_hbm.at[pl.ds(0, WAVE * BLK), :], blk_v.at[p],
            gsem[p]).wait()

    def wait_out(p):
        pltpu.make_async_copy(
            tab_hbm.at[pl.ds(0, WAVE), :], rows_v.at[p],
            osem[p]).wait()

    def extract(w, p):
        def body(g, carry):
            v = idx_s[pl.ds(w * WAVE + g * LANES, LANES)]
            jv = v & (BLK - 1)
            for k in range(LANES):
                i = g * LANES + k
                src = i * BLK + jv[k]
                rows_v[p, i, pl.ds(0, LANES)] = (
                    blk_v[p, src, pl.ds(0, LANES)])
                rows_v[p, i, pl.ds(LANES, LANES)] = (
                    blk_v[p, src, pl.ds(LANES, LANES)])
            return carry

        lax.fori_loop(0, WAVE // LANES, body, 0)

    fire(0, 0)
    for w in range(NWAVE):
        p = w % 2
        if w + 1 < NWAVE:
            fire(w + 1, 1 - p)
        wait_gather(p)
        if w >= 2:
            wait_out(p)
        extract(w, p)
        pltpu.async_copy(rows_v.at[p],
                         out_hbm.at[pl.ds(base + w * WAVE, WAVE), :],
                         osem[p])
    wait_out(0)
    wait_out(1)


def kernel(indices, table):
    return _gather(indices.astype(jnp.int32), table)
